# Initial kernel scaffold; baseline (speedup 1.0000x reference)
import jax
import jax.numpy as jnp
from jax.experimental import pallas as pl


def kernel(x, edge_index, Wl0, Wr0, b0, Ws0, Wl1, Wr1, b1):
    raise NotImplementedError("write your pallas kernel here")


# speedup vs baseline: 5.5225x; 5.5225x over previous
import functools

import jax
import jax.numpy as jnp
from jax import lax
from jax.experimental import pallas as pl
from jax.experimental.pallas import tpu as pltpu
from jax.experimental.pallas import tpu_sc as plsc

_N = 10000
_E = 320000
_NC = 2
_NS = 16
_NW = _NC * _NS
_B = 80
_EPW = _E // _NW
_STEPS = _EPW // _B
_RPT = _N // _NS
_ZR = 125
_WR = 624
_WTAIL = _N - _NS * _WR

_MESH = plsc.VectorSubcoreMesh(core_axis_name="c", subcore_axis_name="s")
_SC_PARAMS = pltpu.CompilerParams(needs_layout_passes=False)


def _zero_fill(buf, rows, width):
    def row(i, carry):
        for j in range(width // 16):
            buf[i, pl.ds(j * 16, 16)] = jnp.zeros((16,), jnp.float32)
        return carry
    lax.fori_loop(0, rows, row, 0)


def _seg0_body(x_hbm, src_hbm, dst_hbm, agg_hbm, cnt_hbm,
               srcv, dstv, rows, zbuf, hist, acc_sh, sem):
    c = lax.axis_index("c")
    s = lax.axis_index("s")
    wid = c * _NS + s

    _zero_fill(zbuf, _ZR, 128)

    def hzero(i, carry):
        hist[pl.ds(i * 16, 16)] = jnp.zeros((16,), jnp.float32)
        return carry
    lax.fori_loop(0, _N // 16, hzero, 0)

    for k in range(_RPT // _ZR):
        off = s * _RPT + k * _ZR
        pltpu.sync_copy(zbuf, acc_sh.at[pl.ds(off, _ZR)])
    plsc.subcore_barrier()

    base0 = wid * _EPW
    ones16 = jnp.ones((16,), jnp.float32)

    def step(i, carry):
        base = base0 + i * _B
        pltpu.sync_copy(src_hbm.at[pl.ds(base, _B)], srcv)
        pltpu.sync_copy(dst_hbm.at[pl.ds(base, _B)], dstv)
        pltpu.async_copy(x_hbm.at[srcv], rows, sem).wait()
        pltpu.sync_copy(rows, acc_sh.at[dstv], add=True)
        for j in range(_B // 16):
            idx = dstv[pl.ds(j * 16, 16)]
            plsc.addupdate_scatter(hist, [idx], ones16)
        return carry
    lax.fori_loop(0, _STEPS, step, 0)
    plsc.subcore_barrier()

    pltpu.sync_copy(hist, cnt_hbm.at[pl.ds(wid * _N, _N)])
    off = s * _WR
    pltpu.sync_copy(acc_sh.at[pl.ds(off, _WR)], agg_hbm.at[c, pl.ds(off, _WR)])

    @pl.when(s == 0)
    def _tail():
        t = _NS * _WR
        pltpu.sync_copy(acc_sh.at[pl.ds(t, _WTAIL)], agg_hbm.at[c, pl.ds(t, _WTAIL)])


_seg0 = functools.partial(
    pl.kernel,
    out_type=[
        jax.ShapeDtypeStruct((_NC, _N, 128), jnp.float32),
        jax.ShapeDtypeStruct((_NW * _N,), jnp.float32),
    ],
    mesh=_MESH,
    scratch_types=[
        pltpu.VMEM((_B,), jnp.int32),
        pltpu.VMEM((_B,), jnp.int32),
        pltpu.VMEM((_B, 128), jnp.float32),
        pltpu.VMEM((_ZR, 128), jnp.float32),
        pltpu.VMEM((_N,), jnp.float32),
        pltpu.VMEM_SHARED((_N, 128), jnp.float32),
        pltpu.SemaphoreType.DMA,
    ],
    compiler_params=_SC_PARAMS,
)(_seg0_body)


def _seg1_body(p_hbm, src_hbm, dst_hbm, agg_hbm,
               srcv, dstv, rows, zbuf, acc_sh, sem):
    c = lax.axis_index("c")
    s = lax.axis_index("s")
    wid = c * _NS + s

    _zero_fill(zbuf, _ZR, 64)
    for k in range(_RPT // _ZR):
        off = s * _RPT + k * _ZR
        pltpu.sync_copy(zbuf, acc_sh.at[pl.ds(off, _ZR)])
    plsc.subcore_barrier()

    base0 = wid * _EPW

    def step(i, carry):
        base = base0 + i * _B
        pltpu.sync_copy(src_hbm.at[pl.ds(base, _B)], srcv)
        pltpu.sync_copy(dst_hbm.at[pl.ds(base, _B)], dstv)
        pltpu.async_copy(p_hbm.at[srcv], rows, sem).wait()
        pltpu.sync_copy(rows, acc_sh.at[dstv], add=True)
        return carry
    lax.fori_loop(0, _STEPS, step, 0)
    plsc.subcore_barrier()

    off = s * _WR
    pltpu.sync_copy(acc_sh.at[pl.ds(off, _WR)], agg_hbm.at[c, pl.ds(off, _WR)])

    @pl.when(s == 0)
    def _tail():
        t = _NS * _WR
        pltpu.sync_copy(acc_sh.at[pl.ds(t, _WTAIL)], agg_hbm.at[c, pl.ds(t, _WTAIL)])


_seg1 = functools.partial(
    pl.kernel,
    out_type=jax.ShapeDtypeStruct((_NC, _N, 64), jnp.float32),
    mesh=_MESH,
    scratch_types=[
        pltpu.VMEM((_B,), jnp.int32),
        pltpu.VMEM((_B,), jnp.int32),
        pltpu.VMEM((_B, 64), jnp.float32),
        pltpu.VMEM((_ZR, 64), jnp.float32),
        pltpu.VMEM_SHARED((_N, 64), jnp.float32),
        pltpu.SemaphoreType.DMA,
    ],
    compiler_params=pltpu.CompilerParams(needs_layout_passes=False,
                                         use_tc_tiling_on_sc=False),
)(_seg1_body)


_BN = 1000


def _deg_body(dT, o):
    ones = jnp.ones((_NW, 1), jnp.float32)
    deg = jnp.dot(dT[...], ones, precision=lax.Precision.HIGHEST)
    o[...] = 1.0 / jnp.maximum(deg, 1.0)


_degk = pl.pallas_call(
    _deg_body,
    out_shape=jax.ShapeDtypeStruct((_N, 1), jnp.float32),
)


def _tc0_body(a0, a1, d, x, wl0, wf, b0, wl1, h_out, p_out):
    inv = d[...]
    mean = (a0[...] + a1[...]) * inv
    h = jnp.dot(mean, wl0[...], precision=lax.Precision.HIGHEST)
    h = h + jnp.dot(x[...], wf[...], precision=lax.Precision.HIGHEST)
    h = jnp.maximum(h + b0[...], 0.0)
    h_out[...] = h
    p_out[...] = jnp.dot(h, wl1[...], precision=lax.Precision.HIGHEST)


def _tc1_body(g0, g1, d, h, wr1, b1, o):
    inv = d[...]
    o[...] = ((g0[...] + g1[...]) * inv
              + jnp.dot(h[...], wr1[...], precision=lax.Precision.HIGHEST)
              + b1[...])


def _row_spec(w):
    return pl.BlockSpec((_BN, w), lambda i: (i, 0))


def _full_spec(r, w):
    return pl.BlockSpec((r, w), lambda i: (0, 0))


_tc0 = pl.pallas_call(
    _tc0_body,
    grid=(_N // _BN,),
    in_specs=[
        _row_spec(128), _row_spec(128),
        _row_spec(1),
        _row_spec(128), _full_spec(128, 128), _full_spec(128, 128),
        _full_spec(1, 128), _full_spec(128, 64),
    ],
    out_specs=[_row_spec(128), _row_spec(64)],
    out_shape=[
        jax.ShapeDtypeStruct((_N, 128), jnp.float32),
        jax.ShapeDtypeStruct((_N, 64), jnp.float32),
    ],
)

_tc1 = pl.pallas_call(
    _tc1_body,
    grid=(_N // _BN,),
    in_specs=[
        _row_spec(64), _row_spec(64),
        _row_spec(1),
        _row_spec(128), _full_spec(128, 64), _full_spec(1, 64),
    ],
    out_specs=_row_spec(64),
    out_shape=jax.ShapeDtypeStruct((_N, 64), jnp.float32),
)


def kernel(x, edge_index, Wl0, Wr0, b0, Ws0, Wl1, Wr1, b1):
    src = edge_index[0]
    dst = edge_index[1]
    wf = Wr0 + Ws0

    agg, cnt = _seg0(x, src, dst)
    invdeg = _degk(cnt.reshape(_NW, _N).T)
    h, p = _tc0(agg[0], agg[1], invdeg, x, Wl0, wf,
                b0.reshape(1, 128), Wl1)
    aggp = _seg1(p, src, dst)
    out = _tc1(aggp[0], aggp[1], invdeg, h, Wr1, b1.reshape(1, 64))
    return out
